# baseline (speedup 1.0000x reference)
import functools

import jax
import jax.numpy as jnp
from jax import lax
from jax.experimental import pallas as pl
from jax.experimental.pallas import tpu as pltpu
from jax.experimental.pallas import tpu_sc as plsc

_NC = 2
_NS = 16
_GK = 400


def _sc_gather(src, tables):
    (E,) = src.shape
    T = len(tables)
    nw = _NC * _NS
    per_w = E // nw
    K = 200
    nchunk = per_w // K
    assert E % nw == 0 and per_w % K == 0 and K % 8 == 0 and nchunk % 2 == 0
    mesh = plsc.VectorSubcoreMesh(core_axis_name="c", subcore_axis_name="s")

    row_types = []
    type_of = []
    for t in tables:
        key = (t.shape[1], t.dtype)
        if key not in row_types:
            row_types.append(key)
        type_of.append(row_types.index(key))

    scratch = [pltpu.VMEM((per_w,), jnp.int32)]
    for w, dt in row_types:
        scratch += [pltpu.VMEM((K, w), dt), pltpu.VMEM((K, w), dt)]
    scratch += [pltpu.SemaphoreType.DMA] * (2 * len(row_types))

    @functools.partial(
        pl.kernel,
        out_type=tuple(jax.ShapeDtypeStruct((E, t.shape[1]), t.dtype)
                       for t in tables),
        mesh=mesh,
        scratch_types=scratch,
    )
    def gather_kernel(*refs):
        src_hbm = refs[0]
        t_hbm = refs[1:1 + T]
        o_hbm = refs[1 + T:1 + 2 * T]
        idx_all = refs[1 + 2 * T]
        nt = len(row_types)
        bufs = refs[2 + 2 * T:2 + 2 * T + 2 * nt]
        sems = refs[2 + 2 * T + 2 * nt:2 + 2 * T + 4 * nt]
        wid = lax.axis_index("s") * _NC + lax.axis_index("c")
        base = wid * per_w
        pltpu.sync_copy(src_hbm.at[pl.ds(base, per_w)], idx_all)

        for t in range(T):
            tb = t_hbm[t]
            ob = o_hbm[t]
            rows = bufs[2 * type_of[t]:2 * type_of[t] + 2]
            sem = sems[2 * type_of[t]:2 * type_of[t] + 2]

            def start(c, b, tb=tb, rows=rows, sem=sem):
                pltpu.async_copy(tb.at[idx_all.at[pl.ds(c * K, K)]],
                                 rows[b], sem[b])

            def finish(c, b, tb=tb, ob=ob, rows=rows, sem=sem):
                pltpu.make_async_copy(tb.at[pl.ds(0, K)], rows[b],
                                      sem[b]).wait()
                pltpu.sync_copy(rows[b], ob.at[pl.ds(base + c * K, K)])

            start(0, 0)

            @pl.loop(0, nchunk - 2, step=2)
            def _(i):
                start(i + 1, 1)
                finish(i, 0)
                start(i + 2, 0)
                finish(i + 1, 1)

            start(nchunk - 1, 1)
            finish(nchunk - 2, 0)
            finish(nchunk - 1, 1)

    return gather_kernel(src, *tables)


def _sc_scatter(dst, msgs, zeros):
    (E,) = dst.shape
    H = msgs[0].shape[1]
    T = len(msgs)
    n_pad = zeros.shape[0]
    per_t = E // _NS
    gk = 160
    nchunk = per_t // gk
    wr = n_pad // 10
    assert (E % _NS == 0 and per_t % gk == 0 and gk % 8 == 0
            and nchunk % 2 == 1 and n_pad % 10 == 0 and wr % 8 == 0)
    mesh = plsc.VectorSubcoreMesh(core_axis_name="c", subcore_axis_name="s")

    @functools.partial(
        pl.kernel,
        out_type=tuple(jax.ShapeDtypeStruct((n_pad, H), jnp.float32)
                       for _ in range(T)),
        mesh=mesh,
        scratch_types=[pltpu.VMEM_SHARED((n_pad, H), jnp.float32),
                       pltpu.VMEM((gk,), jnp.int32),
                       pltpu.VMEM((gk,), jnp.int32),
                       pltpu.VMEM((gk, H), jnp.float32),
                       pltpu.VMEM((gk, H), jnp.float32),
                       pltpu.SemaphoreType.DMA,
                       pltpu.SemaphoreType.DMA,
                       pltpu.SemaphoreType.DMA,
                       pltpu.SemaphoreType.DMA],
    )
    def scatter_kernel(*refs):
        dst_hbm = refs[0]
        m_hbm = refs[1:1 + T]
        z_hbm = refs[1 + T]
        o_hbm = refs[2 + T:2 + 2 * T]
        acc = refs[2 + 2 * T]
        idxb = refs[3 + 2 * T:5 + 2 * T]
        rows = refs[5 + 2 * T:7 + 2 * T]
        isem = refs[7 + 2 * T:9 + 2 * T]
        rsem = refs[9 + 2 * T:11 + 2 * T]
        cid = lax.axis_index("c")
        sid = lax.axis_index("s")
        base = sid * per_t
        sl = pl.ds(sid * wr, wr)

        def one_pass(m, o, core):
            @pl.when((cid == core) & (sid < 10))
            def _():
                pltpu.sync_copy(z_hbm.at[sl], acc.at[sl])

            plsc.subcore_barrier()

            @pl.when(cid == core)
            def _():
                def start(c, b, m=m):
                    off = base + c * gk
                    pltpu.async_copy(dst_hbm.at[pl.ds(off, gk)], idxb[b],
                                     isem[b])
                    pltpu.async_copy(m.at[pl.ds(off, gk)], rows[b], rsem[b])

                def finish(c, b, m=m):
                    pltpu.make_async_copy(dst_hbm.at[pl.ds(0, gk)], idxb[b],
                                          isem[b]).wait()
                    pltpu.make_async_copy(m.at[pl.ds(0, gk)], rows[b],
                                          rsem[b]).wait()
                    pltpu.sync_copy(rows[b], acc.at[idxb[b]], add=True)

                start(0, 0)

                @pl.loop(0, nchunk - 1, step=2)
                def _(i):
                    start(i + 1, 1)
                    finish(i, 0)
                    start(i + 2, 0)
                    finish(i + 1, 1)

                finish(nchunk - 1, 0)

            plsc.subcore_barrier()

            @pl.when((cid == core) & (sid < 10))
            def _():
                pltpu.sync_copy(acc.at[sl], o.at[sl])

        for t in range(T):
            one_pass(m_hbm[t], o_hbm[t], t % _NC)

    return scatter_kernel(dst, *msgs, zeros)


def _pre_body(s, w, b, o):
    o[...] = jnp.dot(s[...], w[...], preferred_element_type=jnp.float32) + b[...]


def _edge_body(gb, rbf, e0, e1, e2, w1r, w2a, w2b, w2c,
               b2a, b2b, b2c, o_s, o0, o1, o2):
    h128 = gb.shape[1] // 4
    g = gb[...]
    zg = g[:, :h128].astype(jnp.float32)
    z = zg + jnp.dot(rbf[...], w1r[...], preferred_element_type=jnp.float32)
    h = z * jax.nn.sigmoid(z)
    a_ss = jnp.dot(h, w2a[...], preferred_element_type=jnp.float32) + b2a[...]
    a_sv = jnp.dot(h, w2b[...], preferred_element_type=jnp.float32) + b2b[...]
    a_vv = jnp.dot(h, w2c[...], preferred_element_type=jnp.float32) + b2c[...]
    o_s[...] = a_ss
    o0[...] = a_sv * e0[...] + a_vv * g[:, h128:2 * h128].astype(jnp.float32)
    o1[...] = a_sv * e1[...] + a_vv * g[:, 2 * h128:3 * h128].astype(jnp.float32)
    o2[...] = a_sv * e2[...] + a_vv * g[:, 3 * h128:].astype(jnp.float32)


def _node_body(s, agg_s, v0, v1, v2, a0, a1, a2, w3a, w3b, b3, w4a, w4b,
               b4a, b4b, gam, bet, ln_o, o0, o1, o2):
    vn0 = v0[...] + a0[...]
    vn1 = v1[...] + a1[...]
    vn2 = v2[...] + a2[...]
    vnorm = jnp.sqrt(vn0 * vn0 + vn1 * vn1 + vn2 * vn2)
    sa = s[...] + agg_s[...]
    z = (jnp.dot(sa, w3a[...], preferred_element_type=jnp.float32)
         + jnp.dot(vnorm, w3b[...], preferred_element_type=jnp.float32)
         + b3[...])
    h = z * jax.nn.sigmoid(z)
    delta = jnp.dot(h, w4a[...], preferred_element_type=jnp.float32) + b4a[...]
    gate = jnp.dot(h, w4b[...], preferred_element_type=jnp.float32) + b4b[...]
    x = s[...] + delta
    mu = jnp.mean(x, axis=-1, keepdims=True)
    var = jnp.mean((x - mu) * (x - mu), axis=-1, keepdims=True)
    ln_o[...] = (x - mu) * jax.lax.rsqrt(var + 1e-5) * gam[...] + bet[...]
    o0[...] = gate * vn0
    o1[...] = gate * vn1
    o2[...] = gate * vn2


def kernel(s, v, edge_rbf, edge_vec_unit, W1, b1, W2, b2, W3, b3, W4, b4,
           gamma, beta, edge_index):
    N, H = s.shape
    E = edge_index.shape[1]
    f32 = jnp.float32

    src = edge_index[0]
    dst = edge_index[1]
    v0 = v[:, 0, :]
    v1 = v[:, 1, :]
    v2 = v[:, 2, :]
    e0 = edge_vec_unit[:, 0:1]
    e1 = edge_vec_unit[:, 1:2]
    e2 = edge_vec_unit[:, 2:3]

    w1s = W1[:, :H].T
    w1r = W1[:, H:].T
    w2a = W2[:H, :].T
    w2b = W2[H:2 * H, :].T
    w2c = W2[2 * H:, :].T
    b2a = b2[None, :H]
    b2b = b2[None, H:2 * H]
    b2c = b2[None, 2 * H:]
    w3a = W3[:, :H].T
    w3b = W3[:, H:].T
    w4a = W4[:H, :].T
    w4b = W4[H:, :].T
    b4a = b4[None, :H]
    b4b = b4[None, H:]

    bn = 1000
    zs = pl.pallas_call(
        _pre_body,
        grid=(N // bn,),
        in_specs=[pl.BlockSpec((bn, H), lambda i: (i, 0)),
                  pl.BlockSpec((H, H), lambda i: (0, 0)),
                  pl.BlockSpec((1, H), lambda i: (0, 0))],
        out_specs=pl.BlockSpec((bn, H), lambda i: (i, 0)),
        out_shape=jax.ShapeDtypeStruct((N, H), f32),
    )(s, w1s, b1[None, :])

    bf16 = jnp.bfloat16
    i32 = jnp.int32
    packed = jnp.concatenate(
        [zs.astype(bf16), v0.astype(bf16), v1.astype(bf16), v2.astype(bf16)],
        axis=1)
    tab = lax.bitcast_convert_type(packed.reshape(N, 2 * H, 2), i32)
    (gbits,) = _sc_gather(src, (tab,))
    gb = lax.bitcast_convert_type(gbits, bf16).reshape(E, 4 * H)

    be = 2000
    blk = lambda r, c: pl.BlockSpec((r, c), lambda i: (i, 0))
    full = lambda r, c: pl.BlockSpec((r, c), lambda i: (0, 0))
    msg = pl.pallas_call(
        _edge_body,
        grid=(E // be,),
        in_specs=[blk(be, 4 * H), blk(be, edge_rbf.shape[1]),
                  blk(be, 1), blk(be, 1), blk(be, 1),
                  full(edge_rbf.shape[1], H),
                  full(H, H), full(H, H), full(H, H),
                  full(1, H), full(1, H), full(1, H)],
        out_specs=[blk(be, H)] * 4,
        out_shape=tuple(jax.ShapeDtypeStruct((E, H), f32) for _ in range(4)),
    )(gb, edge_rbf, e0, e1, e2, w1r, w2a, w2b, w2c, b2a, b2b, b2c)

    zeros = jnp.zeros((N, H), f32)
    agg_s, agg0, agg1, agg2 = _sc_scatter(dst, msg, zeros)

    ln, ov0, ov1, ov2 = pl.pallas_call(
        _node_body,
        grid=(N // bn,),
        in_specs=[blk(bn, H)] * 8 + [full(H, H), full(H, H), full(1, H),
                                     full(H, H), full(H, H), full(1, H),
                                     full(1, H), full(1, H), full(1, H)],
        out_specs=[blk(bn, H)] * 4,
        out_shape=tuple(jax.ShapeDtypeStruct((N, H), f32) for _ in range(4)),
    )(s, agg_s, v0, v1, v2, agg0, agg1, agg2, w3a, w3b, b3[None, :],
      w4a, w4b, b4a, b4b, gamma[None, :], beta[None, :])

    return (ln, jnp.stack([ov0, ov1, ov2], axis=1))


# speedup vs baseline: 2.8489x; 2.8489x over previous
import functools

import jax
import jax.numpy as jnp
from jax import lax
from jax.experimental import pallas as pl
from jax.experimental.pallas import tpu as pltpu
from jax.experimental.pallas import tpu_sc as plsc

_NC = 2
_NS = 16
_GK = 400


def _sc_gather(src, tables):
    (E,) = src.shape
    T = len(tables)
    nw = _NC * _NS
    per_w = E // nw
    K = 200
    nchunk = per_w // K
    assert E % nw == 0 and per_w % K == 0 and K % 8 == 0 and nchunk % 2 == 0
    mesh = plsc.VectorSubcoreMesh(core_axis_name="c", subcore_axis_name="s")

    row_types = []
    type_of = []
    for t in tables:
        key = (t.shape[1], t.dtype)
        if key not in row_types:
            row_types.append(key)
        type_of.append(row_types.index(key))

    scratch = [pltpu.VMEM((per_w,), jnp.int32)]
    for w, dt in row_types:
        scratch += [pltpu.VMEM((K, w), dt), pltpu.VMEM((K, w), dt)]
    scratch += [pltpu.SemaphoreType.DMA] * (2 * len(row_types))

    @functools.partial(
        pl.kernel,
        out_type=tuple(jax.ShapeDtypeStruct((E, t.shape[1]), t.dtype)
                       for t in tables),
        mesh=mesh,
        scratch_types=scratch,
    )
    def gather_kernel(*refs):
        src_hbm = refs[0]
        t_hbm = refs[1:1 + T]
        o_hbm = refs[1 + T:1 + 2 * T]
        idx_all = refs[1 + 2 * T]
        nt = len(row_types)
        bufs = refs[2 + 2 * T:2 + 2 * T + 2 * nt]
        sems = refs[2 + 2 * T + 2 * nt:2 + 2 * T + 4 * nt]
        wid = lax.axis_index("s") * _NC + lax.axis_index("c")
        base = wid * per_w
        pltpu.sync_copy(src_hbm.at[pl.ds(base, per_w)], idx_all)

        for t in range(T):
            tb = t_hbm[t]
            ob = o_hbm[t]
            rows = bufs[2 * type_of[t]:2 * type_of[t] + 2]
            sem = sems[2 * type_of[t]:2 * type_of[t] + 2]

            def start(c, b, tb=tb, rows=rows, sem=sem):
                pltpu.async_copy(tb.at[idx_all.at[pl.ds(c * K, K)]],
                                 rows[b], sem[b])

            def finish(c, b, tb=tb, ob=ob, rows=rows, sem=sem):
                pltpu.make_async_copy(tb.at[pl.ds(0, K)], rows[b],
                                      sem[b]).wait()
                pltpu.sync_copy(rows[b], ob.at[pl.ds(base + c * K, K)])

            start(0, 0)

            @pl.loop(0, nchunk - 2, step=2)
            def _(i):
                start(i + 1, 1)
                finish(i, 0)
                start(i + 2, 0)
                finish(i + 1, 1)

            start(nchunk - 1, 1)
            finish(nchunk - 2, 0)
            finish(nchunk - 1, 1)

    return gather_kernel(src, *tables)


def _sc_scatter(dst, msgs, zeros):
    (E,) = dst.shape
    H = msgs[0].shape[1]
    T = len(msgs)
    n_pad = zeros.shape[0]
    per_t = E // _NS
    gk = 160
    nchunk = per_t // gk
    wr = n_pad // 10
    assert (E % _NS == 0 and per_t % gk == 0 and gk % 8 == 0
            and nchunk % 2 == 1 and n_pad % 10 == 0 and wr % 8 == 0)
    mesh = plsc.VectorSubcoreMesh(core_axis_name="c", subcore_axis_name="s")

    @functools.partial(
        pl.kernel,
        out_type=tuple(jax.ShapeDtypeStruct((n_pad, H), jnp.float32)
                       for _ in range(T)),
        mesh=mesh,
        scratch_types=[pltpu.VMEM_SHARED((n_pad, H), jnp.float32),
                       pltpu.VMEM((gk,), jnp.int32),
                       pltpu.VMEM((gk,), jnp.int32),
                       pltpu.VMEM((gk, H), jnp.float32),
                       pltpu.VMEM((gk, H), jnp.float32),
                       pltpu.SemaphoreType.DMA,
                       pltpu.SemaphoreType.DMA,
                       pltpu.SemaphoreType.DMA,
                       pltpu.SemaphoreType.DMA],
    )
    def scatter_kernel(*refs):
        dst_hbm = refs[0]
        m_hbm = refs[1:1 + T]
        z_hbm = refs[1 + T]
        o_hbm = refs[2 + T:2 + 2 * T]
        acc = refs[2 + 2 * T]
        idxb = refs[3 + 2 * T:5 + 2 * T]
        rows = refs[5 + 2 * T:7 + 2 * T]
        isem = refs[7 + 2 * T:9 + 2 * T]
        rsem = refs[9 + 2 * T:11 + 2 * T]
        cid = lax.axis_index("c")
        sid = lax.axis_index("s")
        base = sid * per_t
        sl = pl.ds(sid * wr, wr)

        def one_pass(m, o, core):
            @pl.when((cid == core) & (sid < 10))
            def _():
                pltpu.sync_copy(z_hbm.at[sl], acc.at[sl])

            plsc.subcore_barrier()

            @pl.when(cid == core)
            def _():
                def start(c, b, m=m):
                    off = base + c * gk
                    pltpu.async_copy(dst_hbm.at[pl.ds(off, gk)], idxb[b],
                                     isem[b])
                    pltpu.async_copy(m.at[pl.ds(off, gk)], rows[b], rsem[b])

                def finish(c, b, m=m):
                    pltpu.make_async_copy(dst_hbm.at[pl.ds(0, gk)], idxb[b],
                                          isem[b]).wait()
                    pltpu.make_async_copy(m.at[pl.ds(0, gk)], rows[b],
                                          rsem[b]).wait()
                    pltpu.sync_copy(rows[b], acc.at[idxb[b]], add=True)

                start(0, 0)

                @pl.loop(0, nchunk - 1, step=2)
                def _(i):
                    start(i + 1, 1)
                    finish(i, 0)
                    start(i + 2, 0)
                    finish(i + 1, 1)

                finish(nchunk - 1, 0)

            plsc.subcore_barrier()

            @pl.when((cid == core) & (sid < 10))
            def _():
                pltpu.sync_copy(acc.at[sl], o.at[sl])

        for t in range(T):
            one_pass(m_hbm[t], o_hbm[t], t % _NC)

    return scatter_kernel(dst, *msgs, zeros)


def _rnd_bf16(x):
    u = lax.bitcast_convert_type(x, jnp.uint32)
    return (u + jnp.uint32(0x7FFF) + ((u >> 16) & jnp.uint32(1))) >> 16


def _pre_body(s, v0, v1, v2, w, b, o):
    h = s.shape[1]
    zs = jnp.dot(s[...], w[...], preferred_element_type=jnp.float32) + b[...]
    wa = _rnd_bf16(zs) | (_rnd_bf16(v0[...]) << 16)
    wb = _rnd_bf16(v1[...]) | (_rnd_bf16(v2[...]) << 16)
    o[:, :h] = lax.bitcast_convert_type(wa, jnp.int32)
    o[:, h:] = lax.bitcast_convert_type(wb, jnp.int32)


def _edge_body(gb, rbf, e0, e1, e2, w1r, w2a, w2b, w2c,
               b2a, b2b, b2c, o_s, o0, o1, o2):
    h128 = gb.shape[1] // 2
    w = lax.bitcast_convert_type(gb[...], jnp.uint32)
    wa = w[:, :h128]
    wb = w[:, h128:]
    unlo = lambda u: lax.bitcast_convert_type(u << 16, jnp.float32)
    unhi = lambda u: lax.bitcast_convert_type(u & jnp.uint32(0xFFFF0000),
                                              jnp.float32)
    z = unlo(wa) + jnp.dot(rbf[...], w1r[...],
                           preferred_element_type=jnp.float32)
    h = z * jax.nn.sigmoid(z)
    a_ss = jnp.dot(h, w2a[...], preferred_element_type=jnp.float32) + b2a[...]
    a_sv = jnp.dot(h, w2b[...], preferred_element_type=jnp.float32) + b2b[...]
    a_vv = jnp.dot(h, w2c[...], preferred_element_type=jnp.float32) + b2c[...]
    o_s[...] = a_ss
    o0[...] = a_sv * e0[...] + a_vv * unhi(wa)
    o1[...] = a_sv * e1[...] + a_vv * unlo(wb)
    o2[...] = a_sv * e2[...] + a_vv * unhi(wb)


def _node_body(s, agg_s, v0, v1, v2, a0, a1, a2, w3a, w3b, b3, w4a, w4b,
               b4a, b4b, gam, bet, ln_o, o0, o1, o2):
    vn0 = v0[...] + a0[...]
    vn1 = v1[...] + a1[...]
    vn2 = v2[...] + a2[...]
    vnorm = jnp.sqrt(vn0 * vn0 + vn1 * vn1 + vn2 * vn2)
    sa = s[...] + agg_s[...]
    z = (jnp.dot(sa, w3a[...], preferred_element_type=jnp.float32)
         + jnp.dot(vnorm, w3b[...], preferred_element_type=jnp.float32)
         + b3[...])
    h = z * jax.nn.sigmoid(z)
    delta = jnp.dot(h, w4a[...], preferred_element_type=jnp.float32) + b4a[...]
    gate = jnp.dot(h, w4b[...], preferred_element_type=jnp.float32) + b4b[...]
    x = s[...] + delta
    mu = jnp.mean(x, axis=-1, keepdims=True)
    var = jnp.mean((x - mu) * (x - mu), axis=-1, keepdims=True)
    ln_o[...] = (x - mu) * jax.lax.rsqrt(var + 1e-5) * gam[...] + bet[...]
    o0[...] = gate * vn0
    o1[...] = gate * vn1
    o2[...] = gate * vn2


def kernel(s, v, edge_rbf, edge_vec_unit, W1, b1, W2, b2, W3, b3, W4, b4,
           gamma, beta, edge_index):
    N, H = s.shape
    E = edge_index.shape[1]
    f32 = jnp.float32

    src = edge_index[0]
    dst = edge_index[1]
    v0 = v[:, 0, :]
    v1 = v[:, 1, :]
    v2 = v[:, 2, :]
    e0 = edge_vec_unit[:, 0:1]
    e1 = edge_vec_unit[:, 1:2]
    e2 = edge_vec_unit[:, 2:3]

    w1s = W1[:, :H].T
    w1r = W1[:, H:].T
    w2a = W2[:H, :].T
    w2b = W2[H:2 * H, :].T
    w2c = W2[2 * H:, :].T
    b2a = b2[None, :H]
    b2b = b2[None, H:2 * H]
    b2c = b2[None, 2 * H:]
    w3a = W3[:, :H].T
    w3b = W3[:, H:].T
    w4a = W4[:H, :].T
    w4b = W4[H:, :].T
    b4a = b4[None, :H]
    b4b = b4[None, H:]

    bn = 1000
    blk = lambda r, c: pl.BlockSpec((r, c), lambda i: (i, 0))
    full = lambda r, c: pl.BlockSpec((r, c), lambda i: (0, 0))
    tab = pl.pallas_call(
        _pre_body,
        grid=(N // bn,),
        in_specs=[blk(bn, H)] * 4 + [full(H, H), full(1, H)],
        out_specs=blk(bn, 2 * H),
        out_shape=jax.ShapeDtypeStruct((N, 2 * H), jnp.int32),
    )(s, v0, v1, v2, w1s, b1[None, :])

    (gb,) = _sc_gather(src, (tab,))

    be = 2000
    msg = pl.pallas_call(
        _edge_body,
        grid=(E // be,),
        in_specs=[blk(be, 2 * H), blk(be, edge_rbf.shape[1]),
                  blk(be, 1), blk(be, 1), blk(be, 1),
                  full(edge_rbf.shape[1], H),
                  full(H, H), full(H, H), full(H, H),
                  full(1, H), full(1, H), full(1, H)],
        out_specs=[blk(be, H)] * 4,
        out_shape=tuple(jax.ShapeDtypeStruct((E, H), f32) for _ in range(4)),
    )(gb, edge_rbf, e0, e1, e2, w1r, w2a, w2b, w2c, b2a, b2b, b2c)

    zeros = jnp.zeros((N, H), f32)
    agg_s, agg0, agg1, agg2 = _sc_scatter(dst, msg, zeros)

    ln, ov0, ov1, ov2 = pl.pallas_call(
        _node_body,
        grid=(N // bn,),
        in_specs=[blk(bn, H)] * 8 + [full(H, H), full(H, H), full(1, H),
                                     full(H, H), full(H, H), full(1, H),
                                     full(1, H), full(1, H), full(1, H)],
        out_specs=[blk(bn, H)] * 4,
        out_shape=tuple(jax.ShapeDtypeStruct((N, H), f32) for _ in range(4)),
    )(s, agg_s, v0, v1, v2, agg0, agg1, agg2, w3a, w3b, b3[None, :],
      w4a, w4b, b4a, b4b, gamma[None, :], beta[None, :])

    return (ln, jnp.stack([ov0, ov1, ov2], axis=1))


# speedup vs baseline: 2.9046x; 1.0195x over previous
import functools

import jax
import jax.numpy as jnp
from jax import lax
from jax.experimental import pallas as pl
from jax.experimental.pallas import tpu as pltpu
from jax.experimental.pallas import tpu_sc as plsc

_NC = 2
_NS = 16


def _chunk_size(per, cap):
    for k in range(cap - cap % 8, 0, -8):
        if per % k == 0:
            return k
    raise ValueError(per)


def _pipe(nchunk, start, finish):
    assert nchunk >= 4
    start(0, 0)

    if nchunk % 2 == 0:
        @pl.loop(0, nchunk - 2, step=2)
        def _(i):
            start(i + 1, 1)
            finish(i, 0)
            start(i + 2, 0)
            finish(i + 1, 1)

        start(nchunk - 1, 1)
        finish(nchunk - 2, 0)
        finish(nchunk - 1, 1)
    else:
        @pl.loop(0, nchunk - 1, step=2)
        def _(i):
            start(i + 1, 1)
            finish(i, 0)
            start(i + 2, 0)
            finish(i + 1, 1)

        finish(nchunk - 1, 0)


def _sc_gather(src, tables):
    (E,) = src.shape
    T = len(tables)
    nw = _NC * _NS
    per_w = E // nw
    K = _chunk_size(per_w, 200)
    nchunk = per_w // K
    assert E % nw == 0
    mesh = plsc.VectorSubcoreMesh(core_axis_name="c", subcore_axis_name="s")

    row_types = []
    type_of = []
    for t in tables:
        key = (t.shape[1], t.dtype)
        if key not in row_types:
            row_types.append(key)
        type_of.append(row_types.index(key))

    scratch = [pltpu.VMEM((per_w,), jnp.int32)]
    for w, dt in row_types:
        scratch += [pltpu.VMEM((K, w), dt), pltpu.VMEM((K, w), dt)]
    scratch += [pltpu.SemaphoreType.DMA] * (2 * len(row_types))

    @functools.partial(
        pl.kernel,
        out_type=tuple(jax.ShapeDtypeStruct((E, t.shape[1]), t.dtype)
                       for t in tables),
        mesh=mesh,
        scratch_types=scratch,
    )
    def gather_kernel(*refs):
        src_hbm = refs[0]
        t_hbm = refs[1:1 + T]
        o_hbm = refs[1 + T:1 + 2 * T]
        idx_all = refs[1 + 2 * T]
        nt = len(row_types)
        bufs = refs[2 + 2 * T:2 + 2 * T + 2 * nt]
        sems = refs[2 + 2 * T + 2 * nt:2 + 2 * T + 4 * nt]
        wid = lax.axis_index("s") * _NC + lax.axis_index("c")
        base = wid * per_w
        pltpu.sync_copy(src_hbm.at[pl.ds(base, per_w)], idx_all)

        for t in range(T):
            tb = t_hbm[t]
            ob = o_hbm[t]
            rows = bufs[2 * type_of[t]:2 * type_of[t] + 2]
            sem = sems[2 * type_of[t]:2 * type_of[t] + 2]

            def start(c, b, tb=tb, rows=rows, sem=sem):
                pltpu.async_copy(tb.at[idx_all.at[pl.ds(c * K, K)]],
                                 rows[b], sem[b])

            def finish(c, b, tb=tb, ob=ob, rows=rows, sem=sem):
                pltpu.make_async_copy(tb.at[pl.ds(0, K)], rows[b],
                                      sem[b]).wait()
                pltpu.sync_copy(rows[b], ob.at[pl.ds(base + c * K, K)])

            _pipe(nchunk, start, finish)

    return gather_kernel(src, *tables)


def _sc_scatter(dst, msgs, zeros):
    (E,) = dst.shape
    H = msgs[0].shape[1]
    T = len(msgs)
    n_pad = zeros.shape[0]
    per_t = E // _NS
    gk = _chunk_size(per_t, 160)
    nchunk = per_t // gk
    wr = n_pad // 10
    assert E % _NS == 0 and n_pad % 10 == 0 and wr % 8 == 0
    mesh = plsc.VectorSubcoreMesh(core_axis_name="c", subcore_axis_name="s")

    @functools.partial(
        pl.kernel,
        out_type=tuple(jax.ShapeDtypeStruct((n_pad, H), jnp.float32)
                       for _ in range(T)),
        mesh=mesh,
        scratch_types=[pltpu.VMEM_SHARED((n_pad, H), jnp.float32),
                       pltpu.VMEM((gk,), jnp.int32),
                       pltpu.VMEM((gk,), jnp.int32),
                       pltpu.VMEM((gk, H), jnp.float32),
                       pltpu.VMEM((gk, H), jnp.float32),
                       pltpu.SemaphoreType.DMA,
                       pltpu.SemaphoreType.DMA,
                       pltpu.SemaphoreType.DMA,
                       pltpu.SemaphoreType.DMA],
    )
    def scatter_kernel(*refs):
        dst_hbm = refs[0]
        m_hbm = refs[1:1 + T]
        z_hbm = refs[1 + T]
        o_hbm = refs[2 + T:2 + 2 * T]
        acc = refs[2 + 2 * T]
        idxb = refs[3 + 2 * T:5 + 2 * T]
        rows = refs[5 + 2 * T:7 + 2 * T]
        isem = refs[7 + 2 * T:9 + 2 * T]
        rsem = refs[9 + 2 * T:11 + 2 * T]
        cid = lax.axis_index("c")
        sid = lax.axis_index("s")
        base = sid * per_t
        sl = pl.ds(sid * wr, wr)

        def one_pass(m, o, core):
            @pl.when((cid == core) & (sid < 10))
            def _():
                pltpu.sync_copy(z_hbm.at[sl], acc.at[sl])

            plsc.subcore_barrier()

            @pl.when(cid == core)
            def _():
                def start(c, b, m=m):
                    off = base + c * gk
                    pltpu.async_copy(dst_hbm.at[pl.ds(off, gk)], idxb[b],
                                     isem[b])
                    pltpu.async_copy(m.at[pl.ds(off, gk)], rows[b], rsem[b])

                def finish(c, b, m=m):
                    pltpu.make_async_copy(dst_hbm.at[pl.ds(0, gk)], idxb[b],
                                          isem[b]).wait()
                    pltpu.make_async_copy(m.at[pl.ds(0, gk)], rows[b],
                                          rsem[b]).wait()
                    pltpu.sync_copy(rows[b], acc.at[idxb[b]], add=True)

                _pipe(nchunk, start, finish)

            plsc.subcore_barrier()

            @pl.when((cid == core) & (sid < 10))
            def _():
                pltpu.sync_copy(acc.at[sl], o.at[sl])

        for t in range(T):
            one_pass(m_hbm[t], o_hbm[t], t % _NC)

    return scatter_kernel(dst, *msgs, zeros)


def _rnd_bf16(x):
    u = lax.bitcast_convert_type(x, jnp.uint32)
    return (u + jnp.uint32(0x7FFF) + ((u >> 16) & jnp.uint32(1))) >> 16


def _pre_body(s, v0, v1, v2, w, b, o):
    h = s.shape[1]
    zs = jnp.dot(s[...], w[...], preferred_element_type=jnp.float32) + b[...]
    wa = _rnd_bf16(zs) | (_rnd_bf16(v0[...]) << 16)
    wb = _rnd_bf16(v1[...]) | (_rnd_bf16(v2[...]) << 16)
    o[:, :h] = lax.bitcast_convert_type(wa, jnp.int32)
    o[:, h:] = lax.bitcast_convert_type(wb, jnp.int32)


def _edge_body(gb, rbf, e0, e1, e2, w1r, w2a, w2b, w2c,
               b2a, b2b, b2c, o_s, o0, o1, o2):
    h128 = gb.shape[1] // 2
    w = lax.bitcast_convert_type(gb[...], jnp.uint32)
    wa = w[:, :h128]
    wb = w[:, h128:]
    unlo = lambda u: lax.bitcast_convert_type(u << 16, jnp.float32)
    unhi = lambda u: lax.bitcast_convert_type(u & jnp.uint32(0xFFFF0000),
                                              jnp.float32)
    z = unlo(wa) + jnp.dot(rbf[...], w1r[...],
                           preferred_element_type=jnp.float32)
    h = z * jax.nn.sigmoid(z)
    a_ss = jnp.dot(h, w2a[...], preferred_element_type=jnp.float32) + b2a[...]
    a_sv = jnp.dot(h, w2b[...], preferred_element_type=jnp.float32) + b2b[...]
    a_vv = jnp.dot(h, w2c[...], preferred_element_type=jnp.float32) + b2c[...]
    o_s[...] = a_ss
    o0[...] = a_sv * e0[...] + a_vv * unhi(wa)
    o1[...] = a_sv * e1[...] + a_vv * unlo(wb)
    o2[...] = a_sv * e2[...] + a_vv * unhi(wb)


def _node_body(s, gs0, gs1, v0, v1, v2, a00, a01, a10, a11, a20, a21,
               w3a, w3b, b3, w4a, w4b, b4a, b4b, gam, bet, ln_o, o0, o1, o2):
    vn0 = v0[...] + a00[...] + a01[...]
    vn1 = v1[...] + a10[...] + a11[...]
    vn2 = v2[...] + a20[...] + a21[...]
    vnorm = jnp.sqrt(vn0 * vn0 + vn1 * vn1 + vn2 * vn2)
    sa = s[...] + gs0[...] + gs1[...]
    z = (jnp.dot(sa, w3a[...], preferred_element_type=jnp.float32)
         + jnp.dot(vnorm, w3b[...], preferred_element_type=jnp.float32)
         + b3[...])
    h = z * jax.nn.sigmoid(z)
    delta = jnp.dot(h, w4a[...], preferred_element_type=jnp.float32) + b4a[...]
    gate = jnp.dot(h, w4b[...], preferred_element_type=jnp.float32) + b4b[...]
    x = s[...] + delta
    mu = jnp.mean(x, axis=-1, keepdims=True)
    var = jnp.mean((x - mu) * (x - mu), axis=-1, keepdims=True)
    ln_o[...] = (x - mu) * jax.lax.rsqrt(var + 1e-5) * gam[...] + bet[...]
    o0[...] = gate * vn0
    o1[...] = gate * vn1
    o2[...] = gate * vn2


def kernel(s, v, edge_rbf, edge_vec_unit, W1, b1, W2, b2, W3, b3, W4, b4,
           gamma, beta, edge_index):
    N, H = s.shape
    E = edge_index.shape[1]
    f32 = jnp.float32

    src = edge_index[0]
    dst = edge_index[1]
    v0 = v[:, 0, :]
    v1 = v[:, 1, :]
    v2 = v[:, 2, :]
    e0 = edge_vec_unit[:, 0:1]
    e1 = edge_vec_unit[:, 1:2]
    e2 = edge_vec_unit[:, 2:3]

    w1s = W1[:, :H].T
    w1r = W1[:, H:].T
    w2a = W2[:H, :].T
    w2b = W2[H:2 * H, :].T
    w2c = W2[2 * H:, :].T
    b2a = b2[None, :H]
    b2b = b2[None, H:2 * H]
    b2c = b2[None, 2 * H:]
    w3a = W3[:, :H].T
    w3b = W3[:, H:].T
    w4a = W4[:H, :].T
    w4b = W4[H:, :].T
    b4a = b4[None, :H]
    b4b = b4[None, H:]

    bn = 1000
    blk = lambda r, c: pl.BlockSpec((r, c), lambda i: (i, 0))
    full = lambda r, c: pl.BlockSpec((r, c), lambda i: (0, 0))
    tab = pl.pallas_call(
        _pre_body,
        grid=(N // bn,),
        in_specs=[blk(bn, H)] * 4 + [full(H, H), full(1, H)],
        out_specs=blk(bn, 2 * H),
        out_shape=jax.ShapeDtypeStruct((N, 2 * H), jnp.int32),
    )(s, v0, v1, v2, w1s, b1[None, :])

    E2 = E // 2
    be = 2000
    zeros = jnp.zeros((N, H), f32)
    aggs = []
    for hh in range(2):
        sl = slice(hh * E2, (hh + 1) * E2)
        (gb,) = _sc_gather(src[sl], (tab,))
        msg = pl.pallas_call(
            _edge_body,
            grid=(E2 // be,),
            in_specs=[blk(be, 2 * H), blk(be, edge_rbf.shape[1]),
                      blk(be, 1), blk(be, 1), blk(be, 1),
                      full(edge_rbf.shape[1], H),
                      full(H, H), full(H, H), full(H, H),
                      full(1, H), full(1, H), full(1, H)],
            out_specs=[blk(be, H)] * 4,
            out_shape=tuple(jax.ShapeDtypeStruct((E2, H), f32)
                            for _ in range(4)),
        )(gb, edge_rbf[sl], e0[sl], e1[sl], e2[sl],
          w1r, w2a, w2b, w2c, b2a, b2b, b2c)
        aggs.append(_sc_scatter(dst[sl], msg, zeros))

    ln, ov0, ov1, ov2 = pl.pallas_call(
        _node_body,
        grid=(N // bn,),
        in_specs=[blk(bn, H)] * 12 + [full(H, H), full(H, H), full(1, H),
                                      full(H, H), full(H, H), full(1, H),
                                      full(1, H), full(1, H), full(1, H)],
        out_specs=[blk(bn, H)] * 4,
        out_shape=tuple(jax.ShapeDtypeStruct((N, H), f32) for _ in range(4)),
    )(s, aggs[0][0], aggs[1][0], v0, v1, v2,
      aggs[0][1], aggs[1][1], aggs[0][2], aggs[1][2], aggs[0][3], aggs[1][3],
      w3a, w3b, b3[None, :],
      w4a, w4b, b4a, b4b, gamma[None, :], beta[None, :])

    return (ln, jnp.stack([ov0, ov1, ov2], axis=1))


# speedup vs baseline: 2.9923x; 1.0302x over previous
import functools

import jax
import jax.numpy as jnp
from jax import lax
from jax.experimental import pallas as pl
from jax.experimental.pallas import tpu as pltpu
from jax.experimental.pallas import tpu_sc as plsc

_NC = 2
_NS = 16


def _chunk_size(per, cap):
    for k in range(cap - cap % 8, 0, -8):
        if per % k == 0:
            return k
    raise ValueError(per)


def _pipe(nchunk, start, finish):
    assert nchunk >= 4
    start(0, 0)

    if nchunk % 2 == 0:
        @pl.loop(0, nchunk - 2, step=2)
        def _(i):
            start(i + 1, 1)
            finish(i, 0)
            start(i + 2, 0)
            finish(i + 1, 1)

        start(nchunk - 1, 1)
        finish(nchunk - 2, 0)
        finish(nchunk - 1, 1)
    else:
        @pl.loop(0, nchunk - 1, step=2)
        def _(i):
            start(i + 1, 1)
            finish(i, 0)
            start(i + 2, 0)
            finish(i + 1, 1)

        finish(nchunk - 1, 0)


def _sc_gather(src, tables):
    (E,) = src.shape
    T = len(tables)
    nw = _NC * _NS
    per_w = E // nw
    K = _chunk_size(per_w, 200)
    nchunk = per_w // K
    assert E % nw == 0
    mesh = plsc.VectorSubcoreMesh(core_axis_name="c", subcore_axis_name="s")

    row_types = []
    type_of = []
    for t in tables:
        key = (t.shape[1], t.dtype)
        if key not in row_types:
            row_types.append(key)
        type_of.append(row_types.index(key))

    scratch = [pltpu.VMEM((per_w,), jnp.int32)]
    for w, dt in row_types:
        scratch += [pltpu.VMEM((K, w), dt), pltpu.VMEM((K, w), dt)]
    scratch += [pltpu.SemaphoreType.DMA] * (2 * len(row_types))

    @functools.partial(
        pl.kernel,
        out_type=tuple(jax.ShapeDtypeStruct((E, t.shape[1]), t.dtype)
                       for t in tables),
        mesh=mesh,
        scratch_types=scratch,
    )
    def gather_kernel(*refs):
        src_hbm = refs[0]
        t_hbm = refs[1:1 + T]
        o_hbm = refs[1 + T:1 + 2 * T]
        idx_all = refs[1 + 2 * T]
        nt = len(row_types)
        bufs = refs[2 + 2 * T:2 + 2 * T + 2 * nt]
        sems = refs[2 + 2 * T + 2 * nt:2 + 2 * T + 4 * nt]
        wid = lax.axis_index("s") * _NC + lax.axis_index("c")
        base = wid * per_w
        pltpu.sync_copy(src_hbm.at[pl.ds(base, per_w)], idx_all)

        for t in range(T):
            tb = t_hbm[t]
            ob = o_hbm[t]
            rows = bufs[2 * type_of[t]:2 * type_of[t] + 2]
            sem = sems[2 * type_of[t]:2 * type_of[t] + 2]

            def start(c, b, tb=tb, rows=rows, sem=sem):
                pltpu.async_copy(tb.at[idx_all.at[pl.ds(c * K, K)]],
                                 rows[b], sem[b])

            def finish(c, b, tb=tb, ob=ob, rows=rows, sem=sem):
                pltpu.make_async_copy(tb.at[pl.ds(0, K)], rows[b],
                                      sem[b]).wait()
                pltpu.sync_copy(rows[b], ob.at[pl.ds(base + c * K, K)])

            _pipe(nchunk, start, finish)

    return gather_kernel(src, *tables)


def _sc_scatter(dst, msgs, zeros):
    (E,) = dst.shape
    H = msgs[0].shape[1]
    T = len(msgs)
    n_pad = zeros.shape[0]
    per_t = E // _NS
    gk = _chunk_size(per_t, 160)
    nchunk = per_t // gk
    wr = n_pad // 10
    assert E % _NS == 0 and n_pad % 10 == 0 and wr % 8 == 0
    mesh = plsc.VectorSubcoreMesh(core_axis_name="c", subcore_axis_name="s")

    @functools.partial(
        pl.kernel,
        out_type=tuple(jax.ShapeDtypeStruct((n_pad, H), jnp.float32)
                       for _ in range(T)),
        mesh=mesh,
        scratch_types=[pltpu.VMEM_SHARED((n_pad, H), jnp.float32),
                       pltpu.VMEM((gk,), jnp.int32),
                       pltpu.VMEM((gk,), jnp.int32),
                       pltpu.VMEM((gk, H), jnp.float32),
                       pltpu.VMEM((gk, H), jnp.float32),
                       pltpu.SemaphoreType.DMA,
                       pltpu.SemaphoreType.DMA,
                       pltpu.SemaphoreType.DMA,
                       pltpu.SemaphoreType.DMA],
    )
    def scatter_kernel(*refs):
        dst_hbm = refs[0]
        m_hbm = refs[1:1 + T]
        z_hbm = refs[1 + T]
        o_hbm = refs[2 + T:2 + 2 * T]
        acc = refs[2 + 2 * T]
        idxb = refs[3 + 2 * T:5 + 2 * T]
        rows = refs[5 + 2 * T:7 + 2 * T]
        isem = refs[7 + 2 * T:9 + 2 * T]
        rsem = refs[9 + 2 * T:11 + 2 * T]
        cid = lax.axis_index("c")
        sid = lax.axis_index("s")
        base = sid * per_t
        sl = pl.ds(sid * wr, wr)

        def one_pass(m, o, core):
            @pl.when((cid == core) & (sid < 10))
            def _():
                pltpu.sync_copy(z_hbm.at[sl], acc.at[sl])

            plsc.subcore_barrier()

            @pl.when(cid == core)
            def _():
                def start(c, b, m=m):
                    off = base + c * gk
                    pltpu.async_copy(dst_hbm.at[pl.ds(off, gk)], idxb[b],
                                     isem[b])
                    pltpu.async_copy(m.at[pl.ds(off, gk)], rows[b], rsem[b])

                def finish(c, b, m=m):
                    pltpu.make_async_copy(dst_hbm.at[pl.ds(0, gk)], idxb[b],
                                          isem[b]).wait()
                    pltpu.make_async_copy(m.at[pl.ds(0, gk)], rows[b],
                                          rsem[b]).wait()
                    pltpu.sync_copy(rows[b], acc.at[idxb[b]], add=True)

                _pipe(nchunk, start, finish)

            plsc.subcore_barrier()

            @pl.when((cid == core) & (sid < 10))
            def _():
                pltpu.sync_copy(acc.at[sl], o.at[sl])

        for t in range(T):
            one_pass(m_hbm[t], o_hbm[t], t % _NC)

    return scatter_kernel(dst, *msgs, zeros)


def _rnd_bf16(x):
    u = lax.bitcast_convert_type(x, jnp.uint32)
    return (u + jnp.uint32(0x7FFF) + ((u >> 16) & jnp.uint32(1))) >> 16


def _pre_body(s, v0, v1, v2, w, b, o):
    h = s.shape[1]
    zs = jnp.dot(s[...], w[...], preferred_element_type=jnp.float32) + b[...]
    wa = _rnd_bf16(zs) | (_rnd_bf16(v0[...]) << 16)
    wb = _rnd_bf16(v1[...]) | (_rnd_bf16(v2[...]) << 16)
    o[:, :h] = lax.bitcast_convert_type(wa, jnp.int32)
    o[:, h:] = lax.bitcast_convert_type(wb, jnp.int32)


def _edge_body(gb, rbf, e0, e1, e2, w1r, w2a, w2b, w2c,
               b2a, b2b, b2c, o_s, o0, o1, o2):
    h128 = gb.shape[1] // 2
    w = lax.bitcast_convert_type(gb[...], jnp.uint32)
    wa = w[:, :h128]
    wb = w[:, h128:]
    unlo = lambda u: lax.bitcast_convert_type(u << 16, jnp.float32)
    unhi = lambda u: lax.bitcast_convert_type(u & jnp.uint32(0xFFFF0000),
                                              jnp.float32)
    z = unlo(wa) + jnp.dot(rbf[...], w1r[...],
                           preferred_element_type=jnp.float32)
    h = z * jax.nn.sigmoid(z)
    a_ss = jnp.dot(h, w2a[...], preferred_element_type=jnp.float32) + b2a[...]
    a_sv = jnp.dot(h, w2b[...], preferred_element_type=jnp.float32) + b2b[...]
    a_vv = jnp.dot(h, w2c[...], preferred_element_type=jnp.float32) + b2c[...]
    o_s[...] = a_ss
    o0[...] = a_sv * e0[...] + a_vv * unhi(wa)
    o1[...] = a_sv * e1[...] + a_vv * unlo(wb)
    o2[...] = a_sv * e2[...] + a_vv * unhi(wb)


def _node_body(s, gs0, gs1, v0, v1, v2, a00, a01, a10, a11, a20, a21,
               w3a, w3b, b3, w4a, w4b, b4a, b4b, gam, bet, ln_o, o0, o1, o2):
    vn0 = v0[...] + a00[...] + a01[...]
    vn1 = v1[...] + a10[...] + a11[...]
    vn2 = v2[...] + a20[...] + a21[...]
    vnorm = jnp.sqrt(vn0 * vn0 + vn1 * vn1 + vn2 * vn2)
    sa = s[...] + gs0[...] + gs1[...]
    z = (jnp.dot(sa, w3a[...], preferred_element_type=jnp.float32)
         + jnp.dot(vnorm, w3b[...], preferred_element_type=jnp.float32)
         + b3[...])
    h = z * jax.nn.sigmoid(z)
    delta = jnp.dot(h, w4a[...], preferred_element_type=jnp.float32) + b4a[...]
    gate = jnp.dot(h, w4b[...], preferred_element_type=jnp.float32) + b4b[...]
    x = s[...] + delta
    mu = jnp.mean(x, axis=-1, keepdims=True)
    var = jnp.mean((x - mu) * (x - mu), axis=-1, keepdims=True)
    ln_o[...] = (x - mu) * jax.lax.rsqrt(var + 1e-5) * gam[...] + bet[...]
    o0[...] = gate * vn0
    o1[...] = gate * vn1
    o2[...] = gate * vn2


def kernel(s, v, edge_rbf, edge_vec_unit, W1, b1, W2, b2, W3, b3, W4, b4,
           gamma, beta, edge_index):
    N, H = s.shape
    E = edge_index.shape[1]
    f32 = jnp.float32

    src = edge_index[0]
    dst = edge_index[1]
    v0 = v[:, 0, :]
    v1 = v[:, 1, :]
    v2 = v[:, 2, :]
    e0 = edge_vec_unit[:, 0:1]
    e1 = edge_vec_unit[:, 1:2]
    e2 = edge_vec_unit[:, 2:3]

    w1s = W1[:, :H].T
    w1r = W1[:, H:].T
    w2a = W2[:H, :].T
    w2b = W2[H:2 * H, :].T
    w2c = W2[2 * H:, :].T
    b2a = b2[None, :H]
    b2b = b2[None, H:2 * H]
    b2c = b2[None, 2 * H:]
    w3a = W3[:, :H].T
    w3b = W3[:, H:].T
    w4a = W4[:H, :].T
    w4b = W4[H:, :].T
    b4a = b4[None, :H]
    b4b = b4[None, H:]

    bn = 1000
    blk = lambda r, c: pl.BlockSpec((r, c), lambda i: (i, 0))
    full = lambda r, c: pl.BlockSpec((r, c), lambda i: (0, 0))
    tab = pl.pallas_call(
        _pre_body,
        grid=(N // bn,),
        in_specs=[blk(bn, H)] * 4 + [full(H, H), full(1, H)],
        out_specs=blk(bn, 2 * H),
        out_shape=jax.ShapeDtypeStruct((N, 2 * H), jnp.int32),
    )(s, v0, v1, v2, w1s, b1[None, :])

    ea = (E * 3 // 5) // 4800 * 4800
    be = 2000
    zeros = jnp.zeros((N, H), f32)
    aggs = []
    for h0, hn in ((0, ea), (ea, E - ea)):
        sl = slice(h0, h0 + hn)
        (gb,) = _sc_gather(src[sl], (tab,))
        msg = pl.pallas_call(
            _edge_body,
            grid=(hn // be,),
            in_specs=[blk(be, 2 * H), blk(be, edge_rbf.shape[1]),
                      blk(be, 1), blk(be, 1), blk(be, 1),
                      full(edge_rbf.shape[1], H),
                      full(H, H), full(H, H), full(H, H),
                      full(1, H), full(1, H), full(1, H)],
            out_specs=[blk(be, H)] * 4,
            out_shape=tuple(jax.ShapeDtypeStruct((hn, H), f32)
                            for _ in range(4)),
        )(gb, edge_rbf[sl], e0[sl], e1[sl], e2[sl],
          w1r, w2a, w2b, w2c, b2a, b2b, b2c)
        aggs.append(_sc_scatter(dst[sl], msg, zeros))

    ln, ov0, ov1, ov2 = pl.pallas_call(
        _node_body,
        grid=(N // bn,),
        in_specs=[blk(bn, H)] * 12 + [full(H, H), full(H, H), full(1, H),
                                      full(H, H), full(H, H), full(1, H),
                                      full(1, H), full(1, H), full(1, H)],
        out_specs=[blk(bn, H)] * 4,
        out_shape=tuple(jax.ShapeDtypeStruct((N, H), f32) for _ in range(4)),
    )(s, aggs[0][0], aggs[1][0], v0, v1, v2,
      aggs[0][1], aggs[1][1], aggs[0][2], aggs[1][2], aggs[0][3], aggs[1][3],
      w3a, w3b, b3[None, :],
      w4a, w4b, b4a, b4b, gamma[None, :], beta[None, :])

    return (ln, jnp.stack([ov0, ov1, ov2], axis=1))
